# Initial kernel scaffold; baseline (speedup 1.0000x reference)
import jax
import jax.numpy as jnp
from jax.experimental import pallas as pl


def kernel(pos, perturbed_pos, node2graph):
    raise NotImplementedError("write your pallas kernel here")


# speedup vs baseline: 4.0705x; 4.0705x over previous
import functools

import jax
import jax.numpy as jnp
from jax import lax
from jax.experimental import pallas as pl
from jax.experimental.pallas import tpu as pltpu
from jax.experimental.pallas import tpu_sc as plsc

_N = 1600000
_G = 100000
_NC, _NS = 2, 16
_NW = _NC * _NS

_FT = 16
_FM = 32
_FD = 24
_L = 80

_C1 = 400
_RPT1 = _N // _NS
_NCH1 = _RPT1 // _C1
_NSUB1 = _C1 // _L

_C2 = 800
_NCH2 = _G // _C2

_C3 = 400
_RPT3 = _N // _NW
_NCH3 = _RPT3 // _C3
_NSUB3 = _C3 // _L

_MESH = plsc.VectorSubcoreMesh(core_axis_name="c", subcore_axis_name="s")
_CPARAMS = pltpu.CompilerParams(use_tc_tiling_on_sc=False,
                                needs_layout_passes=False)


def _lane():
    return lax.broadcasted_iota(jnp.int32, (16,), 0)


def _col(j):
    return jnp.full((16,), j, jnp.int32)


def _rsqrt(q):
    q = jnp.maximum(q, jnp.float32(1e-30))
    i = plsc.bitcast(q, jnp.int32)
    i = jnp.int32(0x5F3759DF) - lax.shift_right_arithmetic(i, 1)
    y = plsc.bitcast(i, jnp.float32)
    for _ in range(3):
        y = y * (jnp.float32(1.5) - jnp.float32(0.5) * q * y * y)
    return y


def _k1_body(posf, pposf, n2g, zeros, mom, table,
             posb0, pposb0, stage0,
             posb1, pposb1, stage1,
             idsb0, idsb1, idsb2, idsb3,
             lsem0, lsem1, ssem0, ssem1):
    c = lax.axis_index("c")
    s = lax.axis_index("s")
    posb = (posb0, posb1)
    pposb = (pposb0, pposb1)
    idsb = (idsb0, idsb1, idsb2, idsb3)
    stage = (stage0, stage1)
    lsem = (lsem0, lsem1)
    ssem = (ssem0, ssem1)

    gslice = _G // _NS
    pltpu.sync_copy(zeros, table.at[pl.ds(s * gslice, gslice), :])
    plsc.subcore_barrier()

    def issue_loads(k, b, b4):
        r0 = s * _RPT1 + k * _C1
        pltpu.async_copy(posf.at[pl.ds(3 * r0, 3 * _C1)], posb[b], lsem[b])
        pltpu.async_copy(pposf.at[pl.ds(3 * r0, 3 * _C1)], pposb[b], lsem[b])
        for j in range(_NSUB1):
            pltpu.async_copy(n2g.at[pl.ds(r0 + j * _L, _L)], idsb[b4].at[j],
                             lsem[b])

    def wait_loads(b, b4):
        pltpu.make_async_copy(posf.at[pl.ds(0, 3 * _C1)], posb[b],
                              lsem[b]).wait()
        pltpu.make_async_copy(pposf.at[pl.ds(0, 3 * _C1)], pposb[b],
                              lsem[b]).wait()
        for j in range(_NSUB1):
            pltpu.make_async_copy(n2g.at[pl.ds(0, _L)], idsb[b4].at[j],
                                  lsem[b]).wait()

    def issue_scatters(b, b4):
        for j in range(_NSUB1):
            pltpu.async_copy(stage[b].at[pl.ds(j * _L, _L), :],
                             table.at[idsb[b4].at[j]], ssem[b], add=True)

    def wait_scatters(b, b4):
        for j in range(_NSUB1):
            pltpu.make_async_copy(stage[b].at[pl.ds(j * _L, _L), :],
                                  table.at[idsb[b4].at[j]], ssem[b]).wait()

    is0 = c == 0

    def compute_stage(b):
        def grp(w, carry):
            rows = w * 16 + _lane()
            p = [plsc.load_gather(pposb[b], [rows * 3 + j]) for j in range(3)]
            x = [plsc.load_gather(posb[b], [rows * 3 + j]) for j in range(3)]
            ones = jnp.full((16,), 1.0, jnp.float32)
            sel = [(ones, p[0] * x[1]),
                   (p[0], p[1] * x[1]),
                   (p[1], p[2] * x[1]),
                   (p[2], p[0] * x[2]),
                   (x[0], p[1] * x[2]),
                   (x[1], p[2] * x[2])]
            for f, (va, vb) in enumerate(sel):
                plsc.store_scatter(stage[b], [rows, _col(f)],
                                   jnp.where(is0, va, vb))
            rest = [x[2],
                    p[0] * p[0], p[0] * p[1], p[0] * p[2],
                    p[1] * p[1], p[1] * p[2], p[2] * p[2],
                    p[0] * x[0], p[1] * x[0], p[2] * x[0]]
            for f, v in enumerate(rest):
                plsc.store_scatter(stage[b], [rows, _col(6 + f)], v)
            return carry

        lax.fori_loop(0, _C1 // 16, grp, 0)

    issue_loads(0, 0, 0)
    issue_loads(1, 1, 1)

    def chunk(k, b, b4):
        wait_loads(b, b4)

        @pl.when(k >= 2)
        def _():
            wait_scatters(b, (b4 + 2) % 4)

        compute_stage(b)
        issue_scatters(b, b4)

        @pl.when(k + 2 < _NCH1)
        def _():
            issue_loads(k + 2, b, (b4 + 2) % 4)

    def quad(t, carry):
        for i in range(4):
            chunk(4 * t + i, i % 2, i)
        return carry

    lax.fori_loop(0, _NCH1 // 4, quad, 0)
    chunk(_NCH1 - 2, 0, 0)
    chunk(_NCH1 - 1, 1, 1)
    wait_scatters(0, 0)
    wait_scatters(1, 1)
    plsc.subcore_barrier()

    pltpu.sync_copy(table.at[pl.ds(s * gslice, gslice), :],
                    mom.at[pl.ds(s * gslice, gslice), pl.ds(c * _FT, _FT)])


_k1 = functools.partial(
    pl.kernel,
    out_type=jax.ShapeDtypeStruct((_G, _FM), jnp.float32),
    mesh=_MESH,
    compiler_params=_CPARAMS,
    scratch_types=[
        pltpu.VMEM_SHARED((_G, _FT), jnp.float32),
        pltpu.VMEM((3 * _C1,), jnp.float32),
        pltpu.VMEM((3 * _C1,), jnp.float32),
        pltpu.VMEM((_C1, _FT), jnp.float32),
        pltpu.VMEM((3 * _C1,), jnp.float32),
        pltpu.VMEM((3 * _C1,), jnp.float32),
        pltpu.VMEM((_C1, _FT), jnp.float32),
        pltpu.VMEM((_NSUB1, _L), jnp.int32),
        pltpu.VMEM((_NSUB1, _L), jnp.int32),
        pltpu.VMEM((_NSUB1, _L), jnp.int32),
        pltpu.VMEM((_NSUB1, _L), jnp.int32),
        pltpu.SemaphoreType.DMA,
        pltpu.SemaphoreType.DMA,
        pltpu.SemaphoreType.DMA,
        pltpu.SemaphoreType.DMA,
    ],
)(_k1_body)


def _k2_body(mom, dtab, fbuf, dbuf):
    c = lax.axis_index("c")
    s = lax.axis_index("s")
    wid = s * _NC + c
    n_u = (_NCH2 - wid + _NW - 1) // _NW

    def chunk(u, carry):
        g0 = (wid + u * _NW) * _C2
        pltpu.sync_copy(mom.at[pl.ds(g0, _C2), :], fbuf)

        def grp(w, cc):
            rows = w * 16 + _lane()

            def ld(f):
                return plsc.load_gather(fbuf, [rows, _col(f)])

            cnt = ld(0)
            sp = [ld(1 + i) for i in range(3)]
            sx = [ld(4 + i) for i in range(3)]
            pp00, pp01, pp02, pp11, pp12, pp22 = (ld(7), ld(8), ld(9), ld(10),
                                                  ld(11), ld(12))
            xp = [[ld(13 + j) for j in range(3)],
                  [ld(16 + j) for j in range(3)],
                  [ld(19 + j) for j in range(3)]]
            inv = jnp.float32(1.0) / jnp.maximum(cnt, jnp.float32(1.0))
            cp = [sp[i] * inv for i in range(3)]
            cx = [sx[i] * inv for i in range(3)]
            ptp00 = pp00 - sp[0] * sp[0] * inv
            ptp01 = pp01 - sp[0] * sp[1] * inv
            ptp02 = pp02 - sp[0] * sp[2] * inv
            ptp11 = pp11 - sp[1] * sp[1] * inv
            ptp12 = pp12 - sp[1] * sp[2] * inv
            ptp22 = pp22 - sp[2] * sp[2] * inv
            otp = [[xp[i][j] - sx[i] * sp[j] * inv for j in range(3)]
                   for i in range(3)]
            q1 = (ptp00 * ptp00 + ptp11 * ptp11 + ptp22 * ptp22
                  + jnp.float32(2.0) * (ptp01 * ptp01 + ptp02 * ptp02
                                        + ptp12 * ptp12))
            q2 = sum(otp[i][j] * otp[i][j] for i in range(3) for j in range(3))
            den = q1 * _rsqrt(q1) + q2 * _rsqrt(q2)
            sneg = jnp.float32(-2.0) / den
            pt = [sneg * ptp00, sneg * ptp01, sneg * ptp02,
                  sneg * ptp11, sneg * ptp12, sneg * ptp22]
            ot = [[-sneg * otp[i][j] for j in range(3)] for i in range(3)]
            ptm = [[pt[0], pt[1], pt[2]],
                   [pt[1], pt[3], pt[4]],
                   [pt[2], pt[4], pt[5]]]
            ccon = [-(cp[0] * ptm[0][j] + cp[1] * ptm[1][j] + cp[2] * ptm[2][j]
                      + cx[0] * ot[0][j] + cx[1] * ot[1][j] + cx[2] * ot[2][j])
                    for j in range(3)]
            vals = pt + [ot[i][j] for i in range(3) for j in range(3)] + ccon
            for f, v in enumerate(vals):
                plsc.store_scatter(dbuf, [rows, _col(f)], v)
            return cc

        lax.fori_loop(0, _C2 // 16, grp, 0)
        pltpu.sync_copy(dbuf, dtab.at[pl.ds(g0, _C2), :])
        return carry

    lax.fori_loop(0, n_u, chunk, 0)


_k2 = functools.partial(
    pl.kernel,
    out_type=jax.ShapeDtypeStruct((_G, _FD), jnp.float32),
    mesh=_MESH,
    compiler_params=_CPARAMS,
    scratch_types=[
        pltpu.VMEM((_C2, _FM), jnp.float32),
        pltpu.VMEM((_C2, _FD), jnp.float32),
    ],
)(_k2_body)


def _k3_body(posf, pposf, n2g, dtab, outf,
             posb0, pposb0, idsb0, consts0, outb0,
             posb1, pposb1, idsb1, consts1, outb1,
             lsem0, lsem1, gsem0, gsem1, osem0, osem1):
    c = lax.axis_index("c")
    s = lax.axis_index("s")
    wid = s * _NC + c
    posb = (posb0, posb1)
    pposb = (pposb0, pposb1)
    idsb = (idsb0, idsb1)
    consts = (consts0, consts1)
    outb = (outb0, outb1)
    lsem = (lsem0, lsem1)
    gsem = (gsem0, gsem1)
    osem = (osem0, osem1)

    def issue_loads(k, b):
        r0 = wid * _RPT3 + k * _C3
        pltpu.async_copy(posf.at[pl.ds(3 * r0, 3 * _C3)], posb[b], lsem[b])
        pltpu.async_copy(pposf.at[pl.ds(3 * r0, 3 * _C3)], pposb[b], lsem[b])
        for j in range(_NSUB3):
            pltpu.async_copy(n2g.at[pl.ds(r0 + j * _L, _L)], idsb[b].at[j],
                             lsem[b])

    def wait_loads(b):
        pltpu.make_async_copy(posf.at[pl.ds(0, 3 * _C3)], posb[b],
                              lsem[b]).wait()
        pltpu.make_async_copy(pposf.at[pl.ds(0, 3 * _C3)], pposb[b],
                              lsem[b]).wait()
        for j in range(_NSUB3):
            pltpu.make_async_copy(n2g.at[pl.ds(0, _L)], idsb[b].at[j],
                                  lsem[b]).wait()

    def phase_a(k, b):
        wait_loads(b)
        for j in range(_NSUB3):
            pltpu.async_copy(dtab.at[idsb[b].at[j]],
                             consts[b].at[pl.ds(j * _L, _L), :], gsem[b])

    def compute_chunk(km1, b):
        @pl.when(km1 >= 2)
        def _():
            pltpu.make_async_copy(outb[b], outf.at[pl.ds(0, 3 * _C3)],
                                  osem[b]).wait()

        for j in range(_NSUB3):
            pltpu.make_async_copy(dtab.at[idsb[b].at[j]],
                                  consts[b].at[pl.ds(j * _L, _L), :],
                                  gsem[b]).wait()

        def grp(w, cc):
            rows = w * 16 + _lane()
            p = [plsc.load_gather(pposb[b], [rows * 3 + j]) for j in range(3)]
            x = [plsc.load_gather(posb[b], [rows * 3 + j]) for j in range(3)]
            cst = [plsc.load_gather(consts[b], [rows, _col(f)])
                   for f in range(18)]
            ptm = [[cst[0], cst[1], cst[2]],
                   [cst[1], cst[3], cst[4]],
                   [cst[2], cst[4], cst[5]]]
            ot = [[cst[6 + 3 * i + j] for j in range(3)] for i in range(3)]
            for j in range(3):
                o = (p[0] * ptm[0][j] + p[1] * ptm[1][j] + p[2] * ptm[2][j]
                     + x[0] * ot[0][j] + x[1] * ot[1][j] + x[2] * ot[2][j]
                     + cst[15 + j])
                plsc.store_scatter(outb[b], [rows * 3 + j], o)
            return cc

        lax.fori_loop(0, _C3 // 16, grp, 0)
        r0 = wid * _RPT3 + km1 * _C3
        pltpu.async_copy(outb[b], outf.at[pl.ds(3 * r0, 3 * _C3)], osem[b])

        @pl.when(km1 + 2 < _NCH3)
        def _():
            issue_loads(km1 + 2, b)

    issue_loads(0, 0)
    issue_loads(1, 1)

    def body(k, carry):
        even = (k % 2) == 0

        @pl.when(even)
        def _():
            phase_a(k, 0)

            @pl.when(k >= 1)
            def _():
                compute_chunk(k - 1, 1)

        @pl.when(jnp.logical_not(even))
        def _():
            phase_a(k, 1)
            compute_chunk(k - 1, 0)

        return carry

    lax.fori_loop(0, _NCH3, body, 0)
    compute_chunk(_NCH3 - 1, (_NCH3 - 1) % 2)
    pltpu.make_async_copy(outb[0], outf.at[pl.ds(0, 3 * _C3)], osem[0]).wait()
    pltpu.make_async_copy(outb[1], outf.at[pl.ds(0, 3 * _C3)], osem[1]).wait()


_k3 = functools.partial(
    pl.kernel,
    out_type=jax.ShapeDtypeStruct((3 * _N,), jnp.float32),
    mesh=_MESH,
    compiler_params=_CPARAMS,
    scratch_types=[
        pltpu.VMEM((3 * _C3,), jnp.float32),
        pltpu.VMEM((3 * _C3,), jnp.float32),
        pltpu.VMEM((_NSUB3, _L), jnp.int32),
        pltpu.VMEM((_C3, _FD), jnp.float32),
        pltpu.VMEM((3 * _C3,), jnp.float32),
        pltpu.VMEM((3 * _C3,), jnp.float32),
        pltpu.VMEM((3 * _C3,), jnp.float32),
        pltpu.VMEM((_NSUB3, _L), jnp.int32),
        pltpu.VMEM((_C3, _FD), jnp.float32),
        pltpu.VMEM((3 * _C3,), jnp.float32),
        pltpu.SemaphoreType.DMA,
        pltpu.SemaphoreType.DMA,
        pltpu.SemaphoreType.DMA,
        pltpu.SemaphoreType.DMA,
        pltpu.SemaphoreType.DMA,
        pltpu.SemaphoreType.DMA,
    ],
)(_k3_body)


def kernel(pos, perturbed_pos, node2graph):
    n2g = node2graph.astype(jnp.int32)
    posf = pos.reshape(-1)
    pposf = perturbed_pos.reshape(-1)
    zeros = jnp.zeros((_G // _NS, _FT), jnp.float32)
    mom = _k1(posf, pposf, n2g, zeros)
    dtab = _k2(mom)
    outf = _k3(posf, pposf, n2g, dtab)
    return outf.reshape(_N, 3)
